# baseline (speedup 1.0000x reference)
import functools

import jax
import jax.numpy as jnp
from jax import lax
from jax.experimental import pallas as pl
from jax.experimental.pallas import tpu as pltpu
from jax.experimental.pallas import tpu_sc as plsc

T, C, E, H = 2048, 768, 8, 2688
HB = 3
HBK = H // HB
BN = 128
NPAD = 4096 + E * BN
NA = 2 * T

_f32 = jnp.float32
_i32 = jnp.int32


def _router_body(x_ref, wr1_ref, br1_ref, wr2_ref, br2_ref, idx_ref, w_ref):
    x = x_ref[...]
    rh = jnp.dot(x, wr1_ref[...], preferred_element_type=_f32)
    rh = jnp.maximum(rh + br1_ref[...], 0.0)
    logits = jnp.dot(rh, wr2_ref[...], preferred_element_type=_f32)
    logits = logits + br2_ref[...]
    m = jnp.max(logits, axis=-1, keepdims=True)
    ex = jnp.exp(logits - m)
    probs = ex / jnp.sum(ex, axis=-1, keepdims=True)
    lane = lax.broadcasted_iota(_i32, probs.shape, 1)
    m1 = jnp.max(probs, axis=-1, keepdims=True)
    i1 = jnp.min(jnp.where(probs == m1, lane, E), axis=-1, keepdims=True)
    pm = jnp.where(lane == i1, -jnp.inf, probs)
    m2 = jnp.max(pm, axis=-1, keepdims=True)
    i2 = jnp.min(jnp.where(pm == m2, lane, E), axis=-1, keepdims=True)
    idx_ref[...] = jnp.where(lane == 0, i1, jnp.where(lane == 1, i2, 0))
    wsum = m1 + m2
    w_ref[...] = jnp.where(lane == 0, m1 / wsum,
                           jnp.where(lane == 1, m2 / wsum, 0.0))


def _router(x2, Wr1, br1, Wr2, br2):
    return pl.pallas_call(
        _router_body,
        out_shape=(jax.ShapeDtypeStruct((T, E), _i32),
                   jax.ShapeDtypeStruct((T, E), _f32)),
    )(x2, Wr1, br1.reshape(1, C // 2), Wr2, br2.reshape(1, E))


_SC_MESH = plsc.VectorSubcoreMesh(core_axis_name="c", subcore_axis_name="s")
_SC_PARAMS = pltpu.CompilerParams(needs_layout_passes=False)
NW = 32
CH = T // NW
CG = CH // 16


def _load_kvecs(idx_hbm, idxw, w, iota):
    pltpu.sync_copy(idx_hbm.at[pl.ds(w * CH, CH), :], idxw)
    zeros = jnp.zeros((16,), _i32)
    ones = zeros + 1
    kvecs = []
    for k in range(2):
        col = zeros if k == 0 else ones
        for g in range(CG):
            kvecs.append(plsc.load_gather(idxw, [iota + g * 16, col]))
    return kvecs


def _hist_body(idx_hbm, hist_hbm, idxw, histv):
    c = lax.axis_index("c")
    s = lax.axis_index("s")
    w = s * 2 + c
    iota = lax.iota(_i32, 16)
    kvecs = _load_kvecs(idx_hbm, idxw, w, iota)
    hist = jnp.zeros((16,), _i32)
    for e in range(E):
        acc = jnp.zeros((16,), _i32)
        for kv in kvecs:
            acc += jnp.where(kv == e, 1, 0)
        hist += jnp.where(iota == e, jnp.sum(acc), 0)
    histv[...] = hist
    pltpu.sync_copy(histv, hist_hbm.at[w])


def _hist(route_idx):
    f = pl.kernel(
        _hist_body,
        out_type=jax.ShapeDtypeStruct((NW, 16), _i32),
        mesh=_SC_MESH,
        scratch_types=[pltpu.VMEM((CH, E), _i32),
                       pltpu.VMEM((16,), _i32)],
        compiler_params=_SC_PARAMS,
    )
    return f(route_idx)


def _sort_body(idx_hbm, hist_hbm, tok_hbm, pos_hbm, off_hbm,
               idxw, tokv, posv, allh, offv):
    c = lax.axis_index("c")
    s = lax.axis_index("s")
    w = s * 2 + c
    iota = lax.iota(_i32, 16)
    kvecs = _load_kvecs(idx_hbm, idxw, w, iota)
    pltpu.sync_copy(hist_hbm, allh)
    totals = jnp.zeros((16,), _i32)
    pre = jnp.zeros((16,), _i32)
    for r in range(NW):
        hv = allh[r, :]
        totals = totals + hv
        pre = pre + jnp.where(r < w, hv, 0)
    rounded = (totals + BN - 1) // BN * BN
    off_incl = plsc.cumsum(rounded)
    off_store = jnp.where(iota < E, off_incl - rounded, off_incl)
    offv[...] = off_store

    @pl.when(w == 0)
    def _():
        pltpu.sync_copy(offv, off_hbm)

    base_vec = (off_incl - rounded) + pre
    carries = [jnp.sum(jnp.where(iota == e, base_vec, 0))
               for e in range(E)]

    for k in range(2):
        for g in range(CG):
            kv = kvecs[k * CG + g]
            pos = jnp.zeros((16,), _i32)
            for e in range(E):
                mi = jnp.where(kv == e, 1, 0)
                cms = plsc.cumsum(mi)
                pos = pos + mi * (carries[e] + cms - 1)
                carries[e] = carries[e] + jnp.sum(mi)
            posv[k, pl.ds(g * 16, 16)] = pos
            tokv[k, pl.ds(g * 16, 16)] = iota + (w * CH + g * 16)
    for k in range(2):
        pltpu.sync_copy(tokv.at[k], tok_hbm.at[posv.at[k]])
        pltpu.sync_copy(posv.at[k], pos_hbm.at[k, pl.ds(w * CH, CH)])


def _sort(route_idx):
    hist = _hist(route_idx)
    f = pl.kernel(
        _sort_body,
        out_type=(jax.ShapeDtypeStruct((NPAD,), _i32),
                  jax.ShapeDtypeStruct((2, T), _i32),
                  jax.ShapeDtypeStruct((16,), _i32)),
        mesh=_SC_MESH,
        scratch_types=[
            pltpu.VMEM((CH, E), _i32),
            pltpu.VMEM((2, CH), _i32),
            pltpu.VMEM((2, CH), _i32),
            pltpu.VMEM((NW, 16), _i32),
            pltpu.VMEM((16,), _i32),
        ],
        compiler_params=_SC_PARAMS,
    )
    return f(route_idx, hist)


GR = NPAD // 32


def _gather_x_body(x_hbm, tok_hbm, xg_hbm, idxv, rows, sem):
    c = lax.axis_index("c")
    s = lax.axis_index("s")
    wid = s * 2 + c
    base = wid * GR
    pltpu.sync_copy(tok_hbm.at[pl.ds(base, GR)], idxv)
    for g in range(GR // 16):
        sl = pl.ds(g * 16, 16)
        idxv[sl] = jnp.minimum(jnp.maximum(idxv[sl], 0), T - 1)
    cp1 = pltpu.async_copy(x_hbm.at[idxv.at[pl.ds(0, 128)]],
                           rows.at[pl.ds(0, 128), :], sem)
    cp2 = pltpu.async_copy(x_hbm.at[idxv.at[pl.ds(128, GR - 128)]],
                           rows.at[pl.ds(128, GR - 128), :], sem)
    cp1.wait()
    cp2.wait()
    pltpu.sync_copy(rows, xg_hbm.at[pl.ds(base, GR), :])


def _gather_x(xpack, tok_sorted):
    f = pl.kernel(
        _gather_x_body,
        out_type=jax.ShapeDtypeStruct((NPAD, C // 2), _i32),
        mesh=_SC_MESH,
        scratch_types=[pltpu.VMEM((GR,), _i32),
                       pltpu.VMEM((GR, C // 2), _i32),
                       pltpu.SemaphoreType.DMA],
        compiler_params=_SC_PARAMS,
    )
    return f(xpack, tok_sorted)


TR = T // 32


def _gather_y_body(yg_hbm, pos_hbm, g0_hbm, g1_hbm, idx0, idx1, r0, r1, sem):
    c = lax.axis_index("c")
    s = lax.axis_index("s")
    wid = s * 2 + c
    base = wid * TR
    i1 = pltpu.async_copy(pos_hbm.at[0, pl.ds(base, TR)], idx0, sem)
    i2 = pltpu.async_copy(pos_hbm.at[1, pl.ds(base, TR)], idx1, sem)
    i1.wait()
    i2.wait()
    cp1 = pltpu.async_copy(yg_hbm.at[idx0], r0, sem)
    cp2 = pltpu.async_copy(yg_hbm.at[idx1], r1, sem)
    cp1.wait()
    cp2.wait()
    w1 = pltpu.async_copy(r0, g0_hbm.at[pl.ds(base, TR), :], sem)
    w2 = pltpu.async_copy(r1, g1_hbm.at[pl.ds(base, TR), :], sem)
    w1.wait()
    w2.wait()


def _gather_y(yg, pos):
    f = pl.kernel(
        _gather_y_body,
        out_type=(jax.ShapeDtypeStruct((T, C // 2), _i32),
                  jax.ShapeDtypeStruct((T, C // 2), _i32)),
        mesh=_SC_MESH,
        scratch_types=[pltpu.VMEM((TR,), _i32),
                       pltpu.VMEM((TR,), _i32),
                       pltpu.VMEM((TR, C // 2), _i32),
                       pltpu.VMEM((TR, C // 2), _i32),
                       pltpu.SemaphoreType.DMA],
        compiler_params=_SC_PARAMS,
    )
    return f(yg, pos)


def _mm_body(off_ref, xg_ref, w1_ref, b1_ref, w2_ref, b2_ref, yg_ref):
    e = pl.program_id(0)
    w1 = w1_ref[0]
    w2 = w2_ref[0]
    b1 = b1_ref[0]
    b2 = b2_ref[0]
    start = off_ref[e]
    nb = (off_ref[e + 1] - start) // BN

    def body_fn(i, _):
        r0 = pl.multiple_of(start + i * BN, BN)
        rows = pl.ds(r0, BN)
        xb = xg_ref[rows, :]
        h = jnp.dot(xb, w1, preferred_element_type=_f32)
        h = jnp.maximum(h + b1, 0.0).astype(jnp.bfloat16)
        part = jnp.dot(h, w2, preferred_element_type=_f32) + b2
        yg_ref[rows, :] = part.astype(jnp.bfloat16)
        return 0

    lax.fori_loop(0, nb, body_fn, 0)


def _grouped_mm(off, xg, W1b, b1, W2b, b2):
    grid_spec = pltpu.PrefetchScalarGridSpec(
        num_scalar_prefetch=1,
        grid=(E,),
        in_specs=[
            pl.BlockSpec((NPAD, C), lambda e, off: (0, 0)),
            pl.BlockSpec((1, C, H), lambda e, off: (e, 0, 0)),
            pl.BlockSpec((1, 1, H), lambda e, off: (e, 0, 0)),
            pl.BlockSpec((1, H, C), lambda e, off: (e, 0, 0)),
            pl.BlockSpec((1, 1, C), lambda e, off: (e, 0, 0)),
        ],
        out_specs=pl.BlockSpec((NPAD, C), lambda e, off: (0, 0)),
    )
    return pl.pallas_call(
        _mm_body,
        grid_spec=grid_spec,
        out_shape=jax.ShapeDtypeStruct((NPAD, C), jnp.bfloat16),
        compiler_params=pltpu.CompilerParams(
            dimension_semantics=("arbitrary",),
        ),
    )(off, xg, W1b, b1.reshape(E, 1, H), W2b, b2.reshape(E, 1, C))


def _combine_body(x_ref, w_ref, g0_ref, g1_ref, out_ref):
    w0 = w_ref[:, 0:1]
    w1 = w_ref[:, 1:2]
    out_ref[...] = (x_ref[...] + w0 * g0_ref[...].astype(_f32)
                    + w1 * g1_ref[...].astype(_f32))


def _combine(x2, route_w, g0, g1):
    return pl.pallas_call(
        _combine_body,
        out_shape=jax.ShapeDtypeStruct((T, C), _f32),
    )(x2, route_w, g0, g1)


def _pack(a):
    return lax.bitcast_convert_type(
        a.reshape(*a.shape[:-1], C // 2, 2), _i32)


def _unpack(a):
    return lax.bitcast_convert_type(a, jnp.bfloat16).reshape(
        *a.shape[:-1], C)


def kernel(x, Wr1, br1, Wr2, br2, W1, b1, W2, b2):
    x2 = x.reshape(T, C)
    xpack = _pack(x2.astype(jnp.bfloat16))
    W1b = W1.astype(jnp.bfloat16)
    W2b = W2.astype(jnp.bfloat16)
    route_idx, route_w = _router(x2, Wr1, br1, Wr2, br2)
    tok_sorted, pos, off = _sort(route_idx)
    xg = _unpack(_gather_x(xpack, tok_sorted))
    yg = _grouped_mm(off, xg, W1b, b1, W2b, b2)
    g0, g1 = _gather_y(_pack(yg), pos)
    out = _combine(x2, route_w, _unpack(g0), _unpack(g1))
    return out.reshape(1, T, C)


# speedup vs baseline: 2.3984x; 2.3984x over previous
import jax
import jax.numpy as jnp
from jax import lax
from jax.experimental import pallas as pl
from jax.experimental.pallas import tpu as pltpu
from jax.experimental.pallas import tpu_sc as plsc

T, C, E, H = 2048, 768, 8, 2688
BN = 128
NPAD = 4096 + E * BN
NW = 32
CH = T // NW
CG = CH // 16

_f32 = jnp.float32
_i32 = jnp.int32


def _router_body(x_ref, wr1_ref, br1_ref, wr2_ref, br2_ref,
                 idx_ref, w_ref, hist_ref):
    x = x_ref[...]
    rh = jnp.dot(x, wr1_ref[...], preferred_element_type=_f32)
    rh = jnp.maximum(rh + br1_ref[...], 0.0)
    logits = jnp.dot(rh, wr2_ref[...], preferred_element_type=_f32)
    logits = logits + br2_ref[...]
    m = jnp.max(logits, axis=-1, keepdims=True)
    ex = jnp.exp(logits - m)
    probs = ex / jnp.sum(ex, axis=-1, keepdims=True)
    lane = lax.broadcasted_iota(_i32, probs.shape, 1)
    m1 = jnp.max(probs, axis=-1, keepdims=True)
    i1 = jnp.min(jnp.where(probs == m1, lane, E), axis=-1, keepdims=True)
    pm = jnp.where(lane == i1, -jnp.inf, probs)
    m2 = jnp.max(pm, axis=-1, keepdims=True)
    i2 = jnp.min(jnp.where(pm == m2, lane, E), axis=-1, keepdims=True)
    idx_ref[...] = jnp.where(lane == 0, i1, jnp.where(lane == 1, i2, 0))
    wsum = m1 + m2
    w_ref[...] = jnp.where(lane == 0, m1 / wsum,
                           jnp.where(lane == 1, m2 / wsum, 0.0))
    lane16 = lax.broadcasted_iota(_i32, (T, 16), 1)
    oh = ((lane16 == i1) | (lane16 == i2)).astype(_i32)
    for j in range(NW):
        seg = oh[j * CH:(j + 1) * CH, :]
        hist_ref[j:j + 1, :] = jnp.sum(seg, axis=0, keepdims=True)


def _router(x2, Wr1, br1, Wr2, br2):
    return pl.pallas_call(
        _router_body,
        out_shape=(jax.ShapeDtypeStruct((T, E), _i32),
                   jax.ShapeDtypeStruct((T, E), _f32),
                   jax.ShapeDtypeStruct((NW, 16), _i32)),
    )(x2, Wr1, br1.reshape(1, C // 2), Wr2, br2.reshape(1, E))


_SC_MESH = plsc.VectorSubcoreMesh(core_axis_name="c", subcore_axis_name="s")
_SC_PARAMS = pltpu.CompilerParams(needs_layout_passes=False)


def _sort_body(idx_hbm, hist_hbm, pos_hbm, off_hbm, idxw, posv, allh, offv):
    c = lax.axis_index("c")
    s = lax.axis_index("s")
    w = s * 2 + c
    iota = lax.iota(_i32, 16)
    pltpu.sync_copy(idx_hbm.at[pl.ds(w * CH, CH), :], idxw)
    zeros = jnp.zeros((16,), _i32)
    ones = zeros + 1
    kvecs = []
    for k in range(2):
        col = zeros if k == 0 else ones
        for g in range(CG):
            kvecs.append(plsc.load_gather(idxw, [iota + g * 16, col]))
    pltpu.sync_copy(hist_hbm, allh)
    totals = jnp.zeros((16,), _i32)
    pre = jnp.zeros((16,), _i32)
    for r in range(NW):
        hv = allh[r, :]
        totals = totals + hv
        pre = pre + jnp.where(r < w, hv, 0)
    rounded = (totals + BN - 1) // BN * BN
    off_incl = plsc.cumsum(rounded)
    off_store = jnp.where(iota < E, off_incl - rounded, off_incl)
    offv[...] = off_store

    @pl.when(w == 0)
    def _():
        pltpu.sync_copy(offv, off_hbm)

    base_vec = (off_incl - rounded) + pre
    carries = [jnp.sum(jnp.where(iota == e, base_vec, 0))
               for e in range(E)]

    for k in range(2):
        for g in range(CG):
            kv = kvecs[k * CG + g]
            pos = jnp.zeros((16,), _i32)
            for e in range(E):
                mi = jnp.where(kv == e, 1, 0)
                cms = plsc.cumsum(mi)
                pos = pos + mi * (carries[e] + cms - 1)
                carries[e] = carries[e] + jnp.sum(mi)
            posv[k, pl.ds(g * 16, 16)] = pos
    for k in range(2):
        pltpu.sync_copy(posv.at[k], pos_hbm.at[k, pl.ds(w * CH, CH)])


def _sort(route_idx, hist):
    f = pl.kernel(
        _sort_body,
        out_type=(jax.ShapeDtypeStruct((2, T), _i32),
                  jax.ShapeDtypeStruct((16,), _i32)),
        mesh=_SC_MESH,
        scratch_types=[
            pltpu.VMEM((CH, E), _i32),
            pltpu.VMEM((2, CH), _i32),
            pltpu.VMEM((NW, 16), _i32),
            pltpu.VMEM((16,), _i32),
        ],
        compiler_params=_SC_PARAMS,
    )
    return f(route_idx, hist)


def _mm_body(off_ref, pos_ref, xb_ref, w1_ref, b1_ref, w2_ref, b2_ref,
             yg_ref):
    e = pl.program_id(0)
    w1 = w1_ref[0]
    w2 = w2_ref[0]
    b1 = b1_ref[0]
    b2 = b2_ref[0]
    p0 = pos_ref[0:1, :]
    p1 = pos_ref[1:2, :]
    xall = xb_ref[...]
    start = off_ref[e]
    nb = (off_ref[e + 1] - start) // BN

    def body_fn(i, _):
        r0 = pl.multiple_of(start + i * BN, BN)
        rowpos = lax.broadcasted_iota(_i32, (BN, T), 0) + r0
        oh = ((rowpos == p0) | (rowpos == p1)).astype(jnp.bfloat16)
        xb = jnp.dot(oh, xall,
                     preferred_element_type=_f32).astype(jnp.bfloat16)
        h = jnp.dot(xb, w1, preferred_element_type=_f32)
        h = jnp.maximum(h + b1, 0.0).astype(jnp.bfloat16)
        part = jnp.dot(h, w2, preferred_element_type=_f32) + b2
        yg_ref[pl.ds(r0, BN), :] = part
        return 0

    lax.fori_loop(0, nb, body_fn, 0)


def _grouped_mm(off, pos, xb16, W1b, b1, W2b, b2):
    grid_spec = pltpu.PrefetchScalarGridSpec(
        num_scalar_prefetch=1,
        grid=(E,),
        in_specs=[
            pl.BlockSpec((2, T), lambda e, off: (0, 0)),
            pl.BlockSpec((T, C), lambda e, off: (0, 0)),
            pl.BlockSpec((1, C, H), lambda e, off: (e, 0, 0)),
            pl.BlockSpec((1, 1, H), lambda e, off: (e, 0, 0)),
            pl.BlockSpec((1, H, C), lambda e, off: (e, 0, 0)),
            pl.BlockSpec((1, 1, C), lambda e, off: (e, 0, 0)),
        ],
        out_specs=pl.BlockSpec((NPAD, C), lambda e, off: (0, 0)),
    )
    return pl.pallas_call(
        _mm_body,
        grid_spec=grid_spec,
        out_shape=jax.ShapeDtypeStruct((NPAD, C), _f32),
        compiler_params=pltpu.CompilerParams(
            dimension_semantics=("arbitrary",),
        ),
    )(off, pos, xb16, W1b, b1.reshape(E, 1, H), W2b, b2.reshape(E, 1, C))


TR = T // 32


def _gather_y_body(yg_hbm, pos_hbm, g0_hbm, g1_hbm, idx0, idx1, r0, r1, sem):
    c = lax.axis_index("c")
    s = lax.axis_index("s")
    wid = s * 2 + c
    base = wid * TR
    i1 = pltpu.async_copy(pos_hbm.at[0, pl.ds(base, TR)], idx0, sem)
    i2 = pltpu.async_copy(pos_hbm.at[1, pl.ds(base, TR)], idx1, sem)
    i1.wait()
    i2.wait()
    cp1 = pltpu.async_copy(yg_hbm.at[idx0], r0, sem)
    cp2 = pltpu.async_copy(yg_hbm.at[idx1], r1, sem)
    cp1.wait()
    cp2.wait()
    w1 = pltpu.async_copy(r0, g0_hbm.at[pl.ds(base, TR), :], sem)
    w2 = pltpu.async_copy(r1, g1_hbm.at[pl.ds(base, TR), :], sem)
    w1.wait()
    w2.wait()


def _gather_y(yg, pos):
    f = pl.kernel(
        _gather_y_body,
        out_type=(jax.ShapeDtypeStruct((T, C), _f32),
                  jax.ShapeDtypeStruct((T, C), _f32)),
        mesh=_SC_MESH,
        scratch_types=[pltpu.VMEM((TR,), _i32),
                       pltpu.VMEM((TR,), _i32),
                       pltpu.VMEM((TR, C), _f32),
                       pltpu.VMEM((TR, C), _f32),
                       pltpu.SemaphoreType.DMA],
        compiler_params=_SC_PARAMS,
    )
    return f(yg, pos)


def _combine_body(x_ref, w_ref, g0_ref, g1_ref, out_ref):
    w0 = w_ref[:, 0:1]
    w1 = w_ref[:, 1:2]
    out_ref[...] = x_ref[...] + w0 * g0_ref[...] + w1 * g1_ref[...]


def _combine(x2, route_w, g0, g1):
    return pl.pallas_call(
        _combine_body,
        out_shape=jax.ShapeDtypeStruct((T, C), _f32),
    )(x2, route_w, g0, g1)


def kernel(x, Wr1, br1, Wr2, br2, W1, b1, W2, b2):
    x2 = x.reshape(T, C)
    xb16 = x2.astype(jnp.bfloat16)
    W1b = W1.astype(jnp.bfloat16)
    W2b = W2.astype(jnp.bfloat16)
    route_idx, route_w, hist = _router(x2, Wr1, br1, Wr2, br2)
    pos, off = _sort(route_idx, hist)
    yg = _grouped_mm(off, pos, xb16, W1b, b1, W2b, b2)
    g0, g1 = _gather_y(yg, pos)
    out = _combine(x2, route_w, g0, g1)
    return out.reshape(1, T, C)


# speedup vs baseline: 2.8167x; 1.1744x over previous
import jax
import jax.numpy as jnp
from jax import lax
from jax.experimental import pallas as pl
from jax.experimental.pallas import tpu as pltpu
from jax.experimental.pallas import tpu_sc as plsc

T, C, E, H = 2048, 768, 8, 2688
BN = 128
NPAD = 4096 + E * BN
NW = 32
CH = T // NW
CG = CH // 16

_f32 = jnp.float32
_i32 = jnp.int32


def _router_body(x_ref, wr1_ref, br1_ref, wr2_ref, br2_ref,
                 idx_ref, w_ref, hist_ref):
    x = x_ref[...]
    rh = jnp.dot(x, wr1_ref[...], preferred_element_type=_f32)
    rh = jnp.maximum(rh + br1_ref[...], 0.0)
    logits = jnp.dot(rh, wr2_ref[...], preferred_element_type=_f32)
    logits = logits + br2_ref[...]
    m = jnp.max(logits, axis=-1, keepdims=True)
    ex = jnp.exp(logits - m)
    probs = ex / jnp.sum(ex, axis=-1, keepdims=True)
    lane = lax.broadcasted_iota(_i32, probs.shape, 1)
    m1 = jnp.max(probs, axis=-1, keepdims=True)
    i1 = jnp.min(jnp.where(probs == m1, lane, E), axis=-1, keepdims=True)
    pm = jnp.where(lane == i1, -jnp.inf, probs)
    m2 = jnp.max(pm, axis=-1, keepdims=True)
    i2 = jnp.min(jnp.where(pm == m2, lane, E), axis=-1, keepdims=True)
    idx_ref[...] = jnp.where(lane == 0, i1, jnp.where(lane == 1, i2, 0))
    wsum = m1 + m2
    w_ref[...] = jnp.where(lane == 0, m1 / wsum,
                           jnp.where(lane == 1, m2 / wsum, 0.0))
    lane16 = lax.broadcasted_iota(_i32, (T, 16), 1)
    oh = ((lane16 == i1) | (lane16 == i2)).astype(_i32)
    for j in range(NW):
        seg = oh[j * CH:(j + 1) * CH, :]
        hist_ref[j:j + 1, :] = jnp.sum(seg, axis=0, keepdims=True)


def _router(x2, Wr1, br1, Wr2, br2):
    return pl.pallas_call(
        _router_body,
        out_shape=(jax.ShapeDtypeStruct((T, E), _i32),
                   jax.ShapeDtypeStruct((T, E), _f32),
                   jax.ShapeDtypeStruct((NW, 16), _i32)),
    )(x2, Wr1, br1.reshape(1, C // 2), Wr2, br2.reshape(1, E))


_SC_MESH = plsc.VectorSubcoreMesh(core_axis_name="c", subcore_axis_name="s")
_SC_PARAMS = pltpu.CompilerParams(needs_layout_passes=False)


def _sort_body(idx_hbm, hist_hbm, pos_hbm, off_hbm,
               idxw, posv, allh, offv):
    c = lax.axis_index("c")
    s = lax.axis_index("s")
    w = s * 2 + c
    iota = lax.iota(_i32, 16)
    pltpu.sync_copy(idx_hbm.at[pl.ds(w * CH, CH), :], idxw)
    zeros = jnp.zeros((16,), _i32)
    ones = zeros + 1
    kvecs = []
    for k in range(2):
        col = zeros if k == 0 else ones
        for g in range(CG):
            kvecs.append(plsc.load_gather(idxw, [iota + g * 16, col]))
    pltpu.sync_copy(hist_hbm, allh)
    totals = jnp.zeros((16,), _i32)
    pre = jnp.zeros((16,), _i32)
    for r in range(NW):
        hv = allh[r, :]
        totals = totals + hv
        pre = pre + jnp.where(r < w, hv, 0)
    rounded = (totals + BN - 1) // BN * BN
    off_incl = plsc.cumsum(rounded)
    off_store = jnp.where(iota < E, off_incl - rounded, off_incl)
    offv[...] = off_store

    @pl.when(w == 0)
    def _():
        pltpu.sync_copy(offv, off_hbm)

    base_vec = (off_incl - rounded) + pre
    carries = [jnp.sum(jnp.where(iota == e, base_vec, 0))
               for e in range(E)]

    for k in range(2):
        for g in range(CG):
            kv = kvecs[k * CG + g]
            pos = jnp.zeros((16,), _i32)
            for e in range(E):
                mi = jnp.where(kv == e, 1, 0)
                cms = plsc.cumsum(mi)
                pos = pos + mi * (carries[e] + cms - 1)
                carries[e] = carries[e] + jnp.sum(mi)
            posv[k, pl.ds(g * 16, 16)] = pos
    for k in range(2):
        pltpu.sync_copy(posv.at[k], pos_hbm.at[k, pl.ds(w * CH, CH)])


def _sort(route_idx, hist):
    f = pl.kernel(
        _sort_body,
        out_type=(jax.ShapeDtypeStruct((2, T), _i32),
                  jax.ShapeDtypeStruct((16,), _i32)),
        mesh=_SC_MESH,
        scratch_types=[
            pltpu.VMEM((CH, E), _i32),
            pltpu.VMEM((2, CH), _i32),
            pltpu.VMEM((NW, 16), _i32),
            pltpu.VMEM((16,), _i32),
        ],
        compiler_params=_SC_PARAMS,
    )
    return f(route_idx, hist)


HB = 3
HBK = H // HB


def _mm_body(off_ref, pos_ref, xb_ref, w1_ref, b1_ref, w2_ref, b2_ref,
             yg_ref, xbg_ref):
    e = pl.program_id(0)
    hb = pl.program_id(1)
    w1 = w1_ref[0].astype(jnp.bfloat16)
    w2 = w2_ref[0].astype(jnp.bfloat16)
    b1 = b1_ref[0]
    b2 = b2_ref[0]
    start = off_ref[e]
    nb = (off_ref[e + 1] - start) // BN

    def body_fn(i, _):
        r0 = pl.multiple_of(start + i * BN, BN)
        rows = pl.ds(r0, BN)

        @pl.when(hb == 0)
        def _():
            p0 = pos_ref[0:1, :]
            p1 = pos_ref[1:2, :]
            rowpos = lax.broadcasted_iota(_i32, (BN, T), 0) + r0
            oh = ((rowpos == p0) | (rowpos == p1)).astype(jnp.bfloat16)
            xbg_ref[rows, :] = jnp.dot(
                oh, xb_ref[...],
                preferred_element_type=_f32).astype(jnp.bfloat16)

        xb = xbg_ref[rows, :]
        h = jnp.dot(xb, w1, preferred_element_type=_f32)
        h = jnp.maximum(h + b1, 0.0).astype(jnp.bfloat16)
        part = jnp.dot(h, w2, preferred_element_type=_f32)

        @pl.when(hb == 0)
        def _():
            yg_ref[rows, :] = part

        @pl.when(hb == 1)
        def _():
            yg_ref[rows, :] += part

        @pl.when(hb == HB - 1)
        def _():
            yg_ref[rows, :] = yg_ref[rows, :] + part + b2
        return 0

    lax.fori_loop(0, nb, body_fn, 0)


def _grouped_mm(off, pos, xb16, W1, b1, W2, b2):
    grid_spec = pltpu.PrefetchScalarGridSpec(
        num_scalar_prefetch=1,
        grid=(E, HB),
        in_specs=[
            pl.BlockSpec((2, T), lambda e, hb, off: (0, 0)),
            pl.BlockSpec((T, C), lambda e, hb, off: (0, 0)),
            pl.BlockSpec((1, C, HBK), lambda e, hb, off: (e, 0, hb)),
            pl.BlockSpec((1, 1, HBK), lambda e, hb, off: (e, 0, hb)),
            pl.BlockSpec((1, HBK, C), lambda e, hb, off: (e, hb, 0)),
            pl.BlockSpec((1, 1, C), lambda e, hb, off: (e, 0, 0)),
        ],
        out_specs=pl.BlockSpec((NPAD, C), lambda e, hb, off: (0, 0)),
        scratch_shapes=[pltpu.VMEM((NPAD, C), jnp.bfloat16)],
    )
    return pl.pallas_call(
        _mm_body,
        grid_spec=grid_spec,
        out_shape=jax.ShapeDtypeStruct((NPAD, C), _f32),
        compiler_params=pltpu.CompilerParams(
            dimension_semantics=("arbitrary", "arbitrary"),
        ),
    )(off, pos, xb16, W1, b1.reshape(E, 1, H), W2, b2.reshape(E, 1, C))


TR = T // 32


def _gather_y_body(yg_hbm, pos_hbm, g0_hbm, g1_hbm, idx0, idx1, r0, r1, sem):
    c = lax.axis_index("c")
    s = lax.axis_index("s")
    wid = s * 2 + c
    base = wid * TR
    i1 = pltpu.async_copy(pos_hbm.at[0, pl.ds(base, TR)], idx0, sem)
    i2 = pltpu.async_copy(pos_hbm.at[1, pl.ds(base, TR)], idx1, sem)
    i1.wait()
    i2.wait()
    cp1 = pltpu.async_copy(yg_hbm.at[idx0], r0, sem)
    cp2 = pltpu.async_copy(yg_hbm.at[idx1], r1, sem)
    cp1.wait()
    cp2.wait()
    w1 = pltpu.async_copy(r0, g0_hbm.at[pl.ds(base, TR), :], sem)
    w2 = pltpu.async_copy(r1, g1_hbm.at[pl.ds(base, TR), :], sem)
    w1.wait()
    w2.wait()


def _gather_y(yg, pos):
    f = pl.kernel(
        _gather_y_body,
        out_type=(jax.ShapeDtypeStruct((T, C), _f32),
                  jax.ShapeDtypeStruct((T, C), _f32)),
        mesh=_SC_MESH,
        scratch_types=[pltpu.VMEM((TR,), _i32),
                       pltpu.VMEM((TR,), _i32),
                       pltpu.VMEM((TR, C), _f32),
                       pltpu.VMEM((TR, C), _f32),
                       pltpu.SemaphoreType.DMA],
        compiler_params=_SC_PARAMS,
    )
    return f(yg, pos)


def _combine_body(x_ref, w_ref, g0_ref, g1_ref, out_ref):
    w0 = w_ref[:, 0:1]
    w1 = w_ref[:, 1:2]
    out_ref[...] = x_ref[...] + w0 * g0_ref[...] + w1 * g1_ref[...]


def _combine(x2, route_w, g0, g1):
    return pl.pallas_call(
        _combine_body,
        out_shape=jax.ShapeDtypeStruct((T, C), _f32),
    )(x2, route_w, g0, g1)


def kernel(x, Wr1, br1, Wr2, br2, W1, b1, W2, b2):
    x2 = x.reshape(T, C)
    xb16 = x2.astype(jnp.bfloat16)
    route_idx, route_w, hist = _router(x2, Wr1, br1, Wr2, br2)
    pos, off = _sort(route_idx, hist)
    yg = _grouped_mm(off, pos, xb16, W1, b1, W2, b2)
    g0, g1 = _gather_y(yg, pos)
    out = _combine(x2, route_w, g0, g1)
    return out.reshape(1, T, C)
